# baseline (speedup 1.0000x reference)
import jax
import jax.numpy as jnp
from jax.experimental import pallas as pl
from jax.experimental.pallas import tpu as pltpu

_P1 = 896
_RA1 = 1024
_RW1 = 1020
_PH = 448
_RA2 = 440
_R2 = 312
_QF = 32
_BT = 8


def _lenet_kernel(xin_ref, w1c_ref, b1_ref, w2c_ref, b2_ref, s2_ref,
                  fc1w_ref, fc1b_ref, fc2w_ref, fc2b_ref, fc3w_ref, fc3b_ref,
                  out_ref, a1_scr, a2_scr):
    f32, bf16 = jnp.float32, jnp.bfloat16
    bt = xin_ref.shape[0]

    def shift_rows(x, k):
        return jnp.concatenate([x[:, k:, :], x[:, :k, :]], axis=1)

    a1_scr[:, pl.ds(_RA1 - 8, 8), :] = jnp.zeros((bt, 8, 40), jnp.bfloat16)
    for j in range(5):
        a1_scr[:, pl.ds(0, _RW1), pl.ds(8 * j, 8)] = xin_ref[:, pl.ds(j, _RW1), :]
    o1p = jnp.dot(a1_scr[...].reshape(bt * _RA1, 40), w1c_ref[...],
                  preferred_element_type=f32).reshape(bt, _RA1, 40)
    o1 = o1p[:, 0:_P1, 0:8]
    for i in range(1, 5):
        o1 = o1 + o1p[:, 32 * i:32 * i + _P1, 8 * i:8 * i + 8]
    o1 = jnp.maximum(o1 + b1_ref[...], 0.0)

    u = jnp.maximum(o1, shift_rows(o1, 1))
    v = u.reshape(bt * 14, 64, 8)
    th = jnp.maximum(v[:, :32, :], v[:, 32:, :])
    th = th.reshape(bt, _PH, 8).astype(bf16)

    for j in range(5):
        a2_scr[:, :, pl.ds(8 * j, 8)] = th[:, 2 * j:2 * j + _RA2, :]
    o2p = jnp.dot(a2_scr[...].reshape(bt * _RA2, 40), w2c_ref[...],
                  preferred_element_type=f32).reshape(bt, _RA2, 80)
    o2 = o2p[:, 0:_R2, 0:16]
    for i in range(1, 5):
        o2 = o2 + o2p[:, 32 * i:32 * i + _R2, 16 * i:16 * i + 16]
    o2 = jnp.maximum(o2 + b2_ref[...], 0.0)

    u2 = jnp.maximum(o2, shift_rows(o2, 2))
    u2 = jnp.maximum(u2, shift_rows(u2, 32))

    comp = jnp.dot(s2_ref[...], u2.reshape(bt * _R2, 16).astype(bf16),
                   preferred_element_type=f32)
    y = jnp.tile(comp, (1, _QF)).reshape(bt, _QF, 16 * _QF)
    lane = jax.lax.broadcasted_iota(jnp.int32, (_QF, 16 * _QF), 1)
    row = jax.lax.broadcasted_iota(jnp.int32, (_QF, 16 * _QF), 0)
    mask = (lane // 16 == row).astype(f32)
    feats = jnp.sum(y * mask[None], axis=1)

    h = jnp.dot(feats.astype(bf16), fc1w_ref[...], preferred_element_type=f32)
    h = jnp.maximum(h + fc1b_ref[...], 0.0)
    h = jnp.dot(h.astype(bf16), fc2w_ref[...], preferred_element_type=f32)
    h = jnp.maximum(h + fc2b_ref[...], 0.0)
    h = jnp.dot(h.astype(bf16), fc3w_ref[...], preferred_element_type=f32)
    out_ref[...] = h + fc3b_ref[...]


def _forward(w1, b1, w2, b2, s2, fc1_w, fc1_b, fc2_w, fc2_b, fc3_w, fc3_b, x):
    b = x.shape[0]
    bt = _BT
    bp = ((b + bt - 1) // bt) * bt
    nb = bp // bt

    xin = jnp.transpose(x, (0, 2, 3, 1))
    xin = jnp.pad(xin, ((0, 0), (0, 0), (0, 0), (0, 5)))
    xin = xin.astype(jnp.bfloat16).reshape(b, 1024, 8)
    if bp != b:
        xin = jnp.pad(xin, ((0, bp - b), (0, 0), (0, 0)))

    w1r = w1.reshape(5, 5, 3, 8)
    w1r = jnp.pad(w1r, ((0, 0), (0, 0), (0, 5), (0, 0)))
    w1c = jnp.transpose(w1r, (1, 2, 0, 3)).reshape(40, 40)
    w2c = jnp.transpose(w2, (1, 0, 2)).reshape(40, 80)

    c2 = lambda i: (0, 0)
    out = pl.pallas_call(
        _lenet_kernel,
        out_shape=jax.ShapeDtypeStruct((bp, 128), jnp.float32),
        grid=(nb,),
        in_specs=[
            pl.BlockSpec((bt, 1024, 8), lambda i: (i, 0, 0)),
            pl.BlockSpec((40, 40), c2),
            pl.BlockSpec((1, 8), c2),
            pl.BlockSpec((40, 80), c2),
            pl.BlockSpec((1, 16), c2),
            pl.BlockSpec((_QF * bt, _R2 * bt), c2),
            pl.BlockSpec((512, 128), c2),
            pl.BlockSpec((1, 128), c2),
            pl.BlockSpec((128, 128), c2),
            pl.BlockSpec((1, 128), c2),
            pl.BlockSpec((128, 128), c2),
            pl.BlockSpec((1, 128), c2),
        ],
        out_specs=pl.BlockSpec((bt, 128), lambda i: (i, 0)),
        scratch_shapes=[pltpu.VMEM((bt, _RA1, 40), jnp.bfloat16),
                        pltpu.VMEM((bt, _RA2, 40), jnp.bfloat16)],
        compiler_params=pltpu.CompilerParams(
            dimension_semantics=("parallel",),
            vmem_limit_bytes=64 * 1024 * 1024),
    )(xin, w1c, b1, w2c, b2, s2, fc1_w, fc1_b, fc2_w, fc2_b, fc3_w, fc3_b)
    return out[:b, :10]


_forward_jit = jax.jit(_forward)


def kernel(w1, b1, w2, b2, s2, fc1_w, fc1_b, fc2_w, fc2_b, fc3_w, fc3_b, x):
    return _forward_jit(w1, b1, w2, b2, s2, fc1_w, fc1_b, fc2_w, fc2_b,
                        fc3_w, fc3_b, x)


# speedup vs baseline: 13.9275x; 13.9275x over previous
import jax
import jax.numpy as jnp
from jax.experimental import pallas as pl
from jax.experimental.pallas import tpu as pltpu

_BT = 32


def _lenet_kernel(xr_ref, wb1_ref, b1t_ref, wb2_ref, b2t_ref, g_ref,
                  fc1b_ref, fc2w_ref, fc2b_ref, fc3w_ref, fc3b_ref,
                  out_ref):
    f32, bf16 = jnp.float32, jnp.bfloat16
    bt = xr_ref.shape[0]

    def shift_rows(x, k):
        return jnp.concatenate([x[:, k:, :], x[:, :k, :]], axis=1)

    def shift_lanes(x, k):
        return jnp.concatenate([x[:, :, k:], x[:, :, :k]], axis=2)

    x2 = xr_ref[...]
    acc = jnp.dot(x2.reshape(bt * 32, 128), wb1_ref[0],
                  preferred_element_type=f32)
    for i in range(1, 5):
        xi = shift_rows(x2, i).reshape(bt * 32, 128)
        acc = acc + jnp.dot(xi, wb1_ref[i], preferred_element_type=f32)
    o1 = jnp.maximum(acc.reshape(bt, 32, 224) + b1t_ref[...], 0.0)

    u = jnp.maximum(o1, shift_rows(o1, 1))
    p1 = jnp.maximum(u, shift_lanes(u, 8)).astype(bf16)

    acc2 = jnp.dot(p1.reshape(bt * 32, 224), wb2_ref[0],
                   preferred_element_type=f32)
    for i in range(1, 5):
        pi = shift_rows(p1, 2 * i).reshape(bt * 32, 224)
        acc2 = acc2 + jnp.dot(pi, wb2_ref[i], preferred_element_type=f32)
    o2 = jnp.maximum(acc2.reshape(bt, 32, 160) + b2t_ref[...], 0.0)

    u2 = jnp.maximum(o2, shift_rows(o2, 2))
    p2 = jnp.maximum(u2, shift_lanes(u2, 16)).astype(bf16)

    h = jnp.dot(p2[:, 0, :], g_ref[0], preferred_element_type=f32)
    for k in range(1, 5):
        h = h + jnp.dot(p2[:, 4 * k, :], g_ref[k], preferred_element_type=f32)
    h = jnp.maximum(h + fc1b_ref[...], 0.0)
    h = jnp.dot(h.astype(bf16), fc2w_ref[...], preferred_element_type=f32)
    h = jnp.maximum(h + fc2b_ref[...], 0.0)
    h = jnp.dot(h.astype(bf16), fc3w_ref[...], preferred_element_type=f32)
    out_ref[...] = h + fc3b_ref[...]


def _forward(w1, b1, w2, b2, s2, fc1_w, fc1_b, fc2_w, fc2_b, fc3_w, fc3_b, x):
    del s2
    f32, bf16 = jnp.float32, jnp.bfloat16
    b = x.shape[0]
    bt = _BT
    bp = ((b + bt - 1) // bt) * bt
    nb = bp // bt

    xin = jnp.transpose(x, (0, 2, 3, 1))
    xin = jnp.pad(xin, ((0, 0), (0, 0), (0, 0), (0, 1)))
    xin = xin.astype(bf16).reshape(b, 32, 128)
    if bp != b:
        xin = jnp.pad(xin, ((0, bp - b), (0, 0), (0, 0)))

    w1r = w1.astype(f32).reshape(5, 5, 3, 8)
    ko = jnp.arange(224)
    w1q, co = ko // 8, ko % 8
    ki = jnp.arange(128)[:, None]
    wb1 = jnp.zeros((5, 128, 224), f32)
    for j in range(5):
        for c in range(3):
            m = (ki == 4 * (w1q[None, :] + j) + c).astype(f32)
            wb1 = wb1 + m[None] * w1r[:, j, c, co][:, None, :]
    wb1 = wb1.astype(bf16)
    b1t = b1[0, co][None, :]

    ko2 = jnp.arange(160)
    w2q, co2 = ko2 // 16, ko2 % 16
    ki2 = jnp.arange(224)[:, None]
    w2f = w2.astype(f32)
    wb2 = jnp.zeros((5, 224, 160), f32)
    for j in range(5):
        for ci in range(6):
            m = (ki2 == 16 * (w2q[None, :] + j) + ci).astype(f32)
            wb2 = wb2 + m[None] * w2f[:, 8 * j + ci, co2][:, None, :]
    wb2 = wb2.astype(bf16)
    b2t = b2[0, co2][None, :]

    ki3 = jnp.arange(160)
    ow2, c3 = ki3 // 32, ki3 % 32
    src = (16 * (5 * jnp.arange(5)[:, None] + ow2[None, :])
           + jnp.minimum(c3, 15)[None, :])
    g = fc1_w[src] * (c3 < 16).astype(bf16)[None, :, None]

    c2 = lambda i: (0, 0)
    c3m = lambda i: (0, 0, 0)
    out = pl.pallas_call(
        _lenet_kernel,
        out_shape=jax.ShapeDtypeStruct((bp, 128), f32),
        grid=(nb,),
        in_specs=[
            pl.BlockSpec((bt, 32, 128), lambda i: (i, 0, 0)),
            pl.BlockSpec((5, 128, 224), c3m),
            pl.BlockSpec((1, 224), c2),
            pl.BlockSpec((5, 224, 160), c3m),
            pl.BlockSpec((1, 160), c2),
            pl.BlockSpec((5, 160, 128), c3m),
            pl.BlockSpec((1, 128), c2),
            pl.BlockSpec((128, 128), c2),
            pl.BlockSpec((1, 128), c2),
            pl.BlockSpec((128, 128), c2),
            pl.BlockSpec((1, 128), c2),
        ],
        out_specs=pl.BlockSpec((bt, 128), lambda i: (i, 0)),
        compiler_params=pltpu.CompilerParams(
            dimension_semantics=("parallel",),
            vmem_limit_bytes=64 * 1024 * 1024),
    )(xin, wb1, b1t, wb2, b2t, g, fc1_b, fc2_w, fc2_b, fc3_w, fc3_b)
    return out[:b, :10]


_forward_jit = jax.jit(_forward)


def kernel(w1, b1, w2, b2, s2, fc1_w, fc1_b, fc2_w, fc2_b, fc3_w, fc3_b, x):
    return _forward_jit(w1, b1, w2, b2, s2, fc1_w, fc1_b, fc2_w, fc2_b,
                        fc3_w, fc3_b, x)


# speedup vs baseline: 14.3800x; 1.0325x over previous
import jax
import jax.numpy as jnp
from jax.experimental import pallas as pl
from jax.experimental.pallas import tpu as pltpu

_BT = 32


def _lenet_kernel(xr_ref, wb1_ref, b1t_ref, wb2_ref, b2t_ref, g_ref,
                  fc1b_ref, fc2w_ref, fc2b_ref, fc3w_ref, fc3b_ref,
                  out_ref):
    f32, bf16 = jnp.float32, jnp.bfloat16
    bt = xr_ref.shape[0]

    def shift_rows(x, k):
        return jnp.concatenate([x[:, k:, :], x[:, :k, :]], axis=1)

    def shift_lanes(x, k):
        return jnp.concatenate([x[:, :, k:], x[:, :, :k]], axis=2)

    x2 = xr_ref[...]
    acc = jnp.dot(x2.reshape(bt * 32, 128), wb1_ref[0],
                  preferred_element_type=f32)
    for i in range(1, 5):
        xi = shift_rows(x2, i).reshape(bt * 32, 128)
        acc = acc + jnp.dot(xi, wb1_ref[i], preferred_element_type=f32)
    o1 = jnp.maximum(acc.reshape(bt, 32, 224) + b1t_ref[...], 0.0)

    u = jnp.maximum(o1, shift_rows(o1, 1))
    p1 = jnp.maximum(u, shift_lanes(u, 8)).astype(bf16)

    acc2 = jnp.dot(p1.reshape(bt * 32, 224), wb2_ref[0],
                   preferred_element_type=f32)
    for i in range(1, 5):
        pi = shift_rows(p1, 2 * i).reshape(bt * 32, 224)
        acc2 = acc2 + jnp.dot(pi, wb2_ref[i], preferred_element_type=f32)
    o2 = jnp.maximum(acc2.reshape(bt, 32, 160) + b2t_ref[...], 0.0)

    u2 = jnp.maximum(o2, shift_rows(o2, 2))
    p2 = jnp.maximum(u2, shift_lanes(u2, 16)).astype(bf16)

    h = jnp.dot(p2[:, 0, :], g_ref[0], preferred_element_type=f32)
    for k in range(1, 5):
        h = h + jnp.dot(p2[:, 4 * k, :], g_ref[k], preferred_element_type=f32)
    h = jnp.maximum(h + fc1b_ref[...], 0.0)
    h = jnp.dot(h.astype(bf16), fc2w_ref[...], preferred_element_type=f32)
    h = jnp.maximum(h + fc2b_ref[...], 0.0)
    h = jnp.dot(h.astype(bf16), fc3w_ref[...], preferred_element_type=f32)
    out_ref[...] = h + fc3b_ref[...]


def _forward(w1, b1, w2, b2, s2, fc1_w, fc1_b, fc2_w, fc2_b, fc3_w, fc3_b, x):
    del s2
    f32, bf16 = jnp.float32, jnp.bfloat16
    b = x.shape[0]
    bt = _BT
    bp = ((b + bt - 1) // bt) * bt
    nb = bp // bt

    xin = jnp.transpose(x, (0, 2, 1, 3))
    xin = jnp.pad(xin, ((0, 0), (0, 0), (0, 1), (0, 0)))
    xin = xin.astype(bf16).reshape(b, 32, 128)
    if bp != b:
        xin = jnp.pad(xin, ((0, bp - b), (0, 0), (0, 0)))

    w1r = w1.astype(f32).reshape(5, 5, 3, 8)
    ko = jnp.arange(224)
    w1q, co = ko // 8, ko % 8
    ki = jnp.arange(128)[:, None]
    wb1 = jnp.zeros((5, 128, 224), f32)
    for j in range(5):
        for c in range(3):
            m = (ki == 32 * c + w1q[None, :] + j).astype(f32)
            wb1 = wb1 + m[None] * w1r[:, j, c, co][:, None, :]
    wb1 = wb1.astype(bf16)
    b1t = b1[0, co][None, :]

    ko2 = jnp.arange(160)
    w2q, co2 = ko2 // 16, ko2 % 16
    ki2 = jnp.arange(224)[:, None]
    w2f = w2.astype(f32)
    wb2 = jnp.zeros((5, 224, 160), f32)
    for j in range(5):
        for ci in range(6):
            m = (ki2 == 16 * (w2q[None, :] + j) + ci).astype(f32)
            wb2 = wb2 + m[None] * w2f[:, 8 * j + ci, co2][:, None, :]
    wb2 = wb2.astype(bf16)
    b2t = b2[0, co2][None, :]

    ki3 = jnp.arange(160)
    ow2, c3 = ki3 // 32, ki3 % 32
    src = (16 * (5 * jnp.arange(5)[:, None] + ow2[None, :])
           + jnp.minimum(c3, 15)[None, :])
    g = fc1_w[src] * (c3 < 16).astype(bf16)[None, :, None]

    c2 = lambda i: (0, 0)
    c3m = lambda i: (0, 0, 0)
    out = pl.pallas_call(
        _lenet_kernel,
        out_shape=jax.ShapeDtypeStruct((bp, 128), f32),
        grid=(nb,),
        in_specs=[
            pl.BlockSpec((bt, 32, 128), lambda i: (i, 0, 0)),
            pl.BlockSpec((5, 128, 224), c3m),
            pl.BlockSpec((1, 224), c2),
            pl.BlockSpec((5, 224, 160), c3m),
            pl.BlockSpec((1, 160), c2),
            pl.BlockSpec((5, 160, 128), c3m),
            pl.BlockSpec((1, 128), c2),
            pl.BlockSpec((128, 128), c2),
            pl.BlockSpec((1, 128), c2),
            pl.BlockSpec((128, 128), c2),
            pl.BlockSpec((1, 128), c2),
        ],
        out_specs=pl.BlockSpec((bt, 128), lambda i: (i, 0)),
        compiler_params=pltpu.CompilerParams(
            dimension_semantics=("parallel",),
            vmem_limit_bytes=64 * 1024 * 1024),
    )(xin, wb1, b1t, wb2, b2t, g, fc1_b, fc2_w, fc2_b, fc3_w, fc3_b)
    return out[:b, :10]


_forward_jit = jax.jit(_forward)


def kernel(w1, b1, w2, b2, s2, fc1_w, fc1_b, fc2_w, fc2_b, fc3_w, fc3_b, x):
    return _forward_jit(w1, b1, w2, b2, s2, fc1_w, fc1_b, fc2_w, fc2_b,
                        fc3_w, fc3_b, x)


# speedup vs baseline: 14.9805x; 1.0418x over previous
import jax
import jax.numpy as jnp
from jax.experimental import pallas as pl
from jax.experimental.pallas import tpu as pltpu

_BT = 64


def _lenet_kernel(xr_ref, wb1_ref, b1t_ref, wb2_ref, b2t_ref, g_ref,
                  fc1b_ref, fc2w_ref, fc2b_ref, fc3w_ref, fc3b_ref,
                  out_ref):
    f32, bf16 = jnp.float32, jnp.bfloat16
    bt = xr_ref.shape[0]

    def shift_rows(x, k):
        return jnp.concatenate([x[:, k:, :], x[:, :k, :]], axis=1)

    def shift_lanes(x, k):
        return jnp.concatenate([x[:, :, k:], x[:, :, :k]], axis=2)

    x2 = xr_ref[...]
    acc = jnp.dot(x2.reshape(bt * 32, 128), wb1_ref[0],
                  preferred_element_type=f32)
    for i in range(1, 5):
        xi = shift_rows(x2, i).reshape(bt * 32, 128)
        acc = acc + jnp.dot(xi, wb1_ref[i], preferred_element_type=f32)
    o1 = jnp.maximum(acc.reshape(bt, 32, 224) + b1t_ref[...], 0.0)

    u = jnp.maximum(o1, shift_rows(o1, 1))
    p1 = jnp.maximum(u, shift_lanes(u, 8)).astype(bf16)

    acc2 = jnp.dot(p1.reshape(bt * 32, 224), wb2_ref[0],
                   preferred_element_type=f32)
    for i in range(1, 5):
        pi = shift_rows(p1, 2 * i).reshape(bt * 32, 224)
        acc2 = acc2 + jnp.dot(pi, wb2_ref[i], preferred_element_type=f32)
    o2 = jnp.maximum(acc2.reshape(bt, 32, 160) + b2t_ref[...], 0.0)

    u2 = jnp.maximum(o2, shift_rows(o2, 2))
    p2 = jnp.maximum(u2, shift_lanes(u2, 16)).astype(bf16)

    h = jnp.dot(p2[:, 0, :], g_ref[0], preferred_element_type=f32)
    for k in range(1, 5):
        h = h + jnp.dot(p2[:, 4 * k, :], g_ref[k], preferred_element_type=f32)
    h = jnp.maximum(h + fc1b_ref[...], 0.0)
    h = jnp.dot(h.astype(bf16), fc2w_ref[...], preferred_element_type=f32)
    h = jnp.maximum(h + fc2b_ref[...], 0.0)
    h = jnp.dot(h.astype(bf16), fc3w_ref[...], preferred_element_type=f32)
    out_ref[...] = h + fc3b_ref[...]


def _forward(w1, b1, w2, b2, s2, fc1_w, fc1_b, fc2_w, fc2_b, fc3_w, fc3_b, x):
    del s2
    f32, bf16 = jnp.float32, jnp.bfloat16
    b = x.shape[0]
    bt = _BT
    bp = ((b + bt - 1) // bt) * bt
    nb = bp // bt

    xin = jnp.transpose(x, (0, 2, 1, 3))
    xin = jnp.pad(xin, ((0, 0), (0, 0), (0, 1), (0, 0)))
    xin = xin.astype(bf16).reshape(b, 32, 128)
    if bp != b:
        xin = jnp.pad(xin, ((0, bp - b), (0, 0), (0, 0)))

    w1r = w1.astype(f32).reshape(5, 5, 3, 8)
    ko = jnp.arange(224)
    w1q, co = ko // 8, ko % 8
    ki = jnp.arange(128)[:, None]
    wb1 = jnp.zeros((5, 128, 224), f32)
    for j in range(5):
        for c in range(3):
            m = (ki == 32 * c + w1q[None, :] + j).astype(f32)
            wb1 = wb1 + m[None] * w1r[:, j, c, co][:, None, :]
    wb1 = wb1.astype(bf16)
    b1t = b1[0, co][None, :]

    ko2 = jnp.arange(160)
    w2q, co2 = ko2 // 16, ko2 % 16
    ki2 = jnp.arange(224)[:, None]
    w2f = w2.astype(f32)
    wb2 = jnp.zeros((5, 224, 160), f32)
    for j in range(5):
        for ci in range(6):
            m = (ki2 == 16 * (w2q[None, :] + j) + ci).astype(f32)
            wb2 = wb2 + m[None] * w2f[:, 8 * j + ci, co2][:, None, :]
    wb2 = wb2.astype(bf16)
    b2t = b2[0, co2][None, :]

    ki3 = jnp.arange(160)
    ow2, c3 = ki3 // 32, ki3 % 32
    src = (16 * (5 * jnp.arange(5)[:, None] + ow2[None, :])
           + jnp.minimum(c3, 15)[None, :])
    g = fc1_w[src] * (c3 < 16).astype(bf16)[None, :, None]

    c2 = lambda i: (0, 0)
    c3m = lambda i: (0, 0, 0)
    out = pl.pallas_call(
        _lenet_kernel,
        out_shape=jax.ShapeDtypeStruct((bp, 128), f32),
        grid=(nb,),
        in_specs=[
            pl.BlockSpec((bt, 32, 128), lambda i: (i, 0, 0)),
            pl.BlockSpec((5, 128, 224), c3m),
            pl.BlockSpec((1, 224), c2),
            pl.BlockSpec((5, 224, 160), c3m),
            pl.BlockSpec((1, 160), c2),
            pl.BlockSpec((5, 160, 128), c3m),
            pl.BlockSpec((1, 128), c2),
            pl.BlockSpec((128, 128), c2),
            pl.BlockSpec((1, 128), c2),
            pl.BlockSpec((128, 128), c2),
            pl.BlockSpec((1, 128), c2),
        ],
        out_specs=pl.BlockSpec((bt, 128), lambda i: (i, 0)),
        compiler_params=pltpu.CompilerParams(
            dimension_semantics=("parallel",),
            vmem_limit_bytes=64 * 1024 * 1024),
    )(xin, wb1, b1t, wb2, b2t, g, fc1_b, fc2_w, fc2_b, fc3_w, fc3_b)
    return out[:b, :10]


_forward_jit = jax.jit(_forward)


def kernel(w1, b1, w2, b2, s2, fc1_w, fc1_b, fc2_w, fc2_b, fc3_w, fc3_b, x):
    return _forward_jit(w1, b1, w2, b2, s2, fc1_w, fc1_b, fc2_w, fc2_b,
                        fc3_w, fc3_b, x)


# speedup vs baseline: 17.7882x; 1.1874x over previous
import jax
import jax.numpy as jnp
from jax.experimental import pallas as pl
from jax.experimental.pallas import tpu as pltpu

_BT = 64


def _lenet_kernel(xc_ref, e_ref, wb1_ref, b1t_ref, wb2_ref, b2t_ref, g_ref,
                  fc1b_ref, fc2w_ref, fc2b_ref, fc3w_ref, fc3b_ref,
                  out_ref):
    f32, bf16 = jnp.float32, jnp.bfloat16
    bt = xc_ref.shape[0]

    def shift_rows(x, k):
        return jnp.concatenate([x[:, k:, :], x[:, :k, :]], axis=1)

    def shift_lanes(x, k):
        return jnp.concatenate([x[:, :, k:], x[:, :, :k]], axis=2)

    xs = xc_ref[...]
    x2 = jnp.dot(xs[:, 0].reshape(bt * 32, 32), e_ref[0],
                 preferred_element_type=f32)
    for c in range(1, 3):
        x2 = x2 + jnp.dot(xs[:, c].reshape(bt * 32, 32), e_ref[c],
                          preferred_element_type=f32)
    x2 = x2.astype(bf16).reshape(bt, 32, 128)

    xbig = jnp.concatenate(
        [x2] + [shift_rows(x2, i) for i in range(1, 5)], axis=2)
    acc = jnp.dot(xbig.reshape(bt * 32, 640), wb1_ref[...],
                  preferred_element_type=f32)
    o1 = jnp.maximum(acc.reshape(bt, 32, 224) + b1t_ref[...], 0.0)

    u = jnp.maximum(o1, shift_rows(o1, 1))
    p1 = jnp.maximum(u, shift_lanes(u, 8)).astype(bf16)

    zpad = jnp.zeros((bt, 32, 32), bf16)
    slabs = []
    for i in range(5):
        slabs.append(shift_rows(p1, 2 * i) if i else p1)
        slabs.append(zpad)
    pbig = jnp.concatenate(slabs, axis=2)
    acc2 = jnp.dot(pbig.reshape(bt * 32, 1280), wb2_ref[...],
                   preferred_element_type=f32)
    o2 = jnp.maximum(acc2.reshape(bt, 32, 160) + b2t_ref[...], 0.0)

    u2 = jnp.maximum(o2, shift_rows(o2, 2))
    p2 = jnp.maximum(u2, shift_lanes(u2, 16)).astype(bf16)

    h = jnp.dot(p2[:, 0, :], g_ref[0], preferred_element_type=f32)
    for k in range(1, 5):
        h = h + jnp.dot(p2[:, 4 * k, :], g_ref[k], preferred_element_type=f32)
    h = jnp.maximum(h + fc1b_ref[...], 0.0)
    h = jnp.dot(h.astype(bf16), fc2w_ref[...], preferred_element_type=f32)
    h = jnp.maximum(h + fc2b_ref[...], 0.0)
    h = jnp.dot(h.astype(bf16), fc3w_ref[...], preferred_element_type=f32)
    out_ref[...] = h + fc3b_ref[...]


def _forward(w1, b1, w2, b2, s2, fc1_w, fc1_b, fc2_w, fc2_b, fc3_w, fc3_b, x):
    del s2
    f32, bf16 = jnp.float32, jnp.bfloat16
    b = x.shape[0]
    bt = _BT
    bp = ((b + bt - 1) // bt) * bt
    nb = bp // bt

    xc = x.astype(bf16)
    if bp != b:
        xc = jnp.pad(xc, ((0, bp - b), (0, 0), (0, 0), (0, 0)))

    e = (jnp.arange(128)[None, None, :]
         == 32 * jnp.arange(3)[:, None, None]
         + jnp.arange(32)[None, :, None]).astype(bf16)

    w1r = w1.astype(f32).reshape(5, 5, 3, 8)
    wb1 = jnp.zeros((5, 3, 32, 28, 8), f32)
    for j in range(5):
        band = (jnp.arange(32)[:, None] == jnp.arange(28)[None, :] + j)
        wb1 = wb1 + (w1r[:, j][:, :, None, None, :]
                     * band[None, None, :, :, None])
    wb1 = jnp.pad(wb1, ((0, 0), (0, 1), (0, 0), (0, 0), (0, 0)))
    wb1 = wb1.reshape(640, 224).astype(bf16)
    b1t = jnp.tile(b1, (1, 28))

    w2f = w2.astype(f32).reshape(5, 5, 8, 16)
    wb2 = jnp.zeros((5, 14, 16, 10, 16), f32)
    for j in range(5):
        band = (jnp.arange(14)[:, None] == jnp.arange(10)[None, :] + j)
        w2j = jnp.pad(w2f[:, j], ((0, 0), (0, 8), (0, 0)))
        wb2 = wb2 + (w2j[:, None, :, None, :]
                     * band[None, :, None, :, None])
    wb2 = wb2.reshape(5, 224, 160)
    wb2 = jnp.pad(wb2, ((0, 0), (0, 32), (0, 0)))
    wb2 = wb2.reshape(1280, 160).astype(bf16)
    b2t = jnp.tile(b2, (1, 10))

    g = fc1_w[:400].reshape(5, 5, 16, 128)
    g = jnp.pad(g, ((0, 0), (0, 0), (0, 16), (0, 0))).reshape(5, 160, 128)

    c2 = lambda i: (0, 0)
    c3m = lambda i: (0, 0, 0)
    out = pl.pallas_call(
        _lenet_kernel,
        out_shape=jax.ShapeDtypeStruct((bp, 128), f32),
        grid=(nb,),
        in_specs=[
            pl.BlockSpec((bt, 3, 32, 32), lambda i: (i, 0, 0, 0)),
            pl.BlockSpec((3, 32, 128), c3m),
            pl.BlockSpec((640, 224), c2),
            pl.BlockSpec((1, 224), c2),
            pl.BlockSpec((1280, 160), c2),
            pl.BlockSpec((1, 160), c2),
            pl.BlockSpec((5, 160, 128), c3m),
            pl.BlockSpec((1, 128), c2),
            pl.BlockSpec((128, 128), c2),
            pl.BlockSpec((1, 128), c2),
            pl.BlockSpec((128, 128), c2),
            pl.BlockSpec((1, 128), c2),
        ],
        out_specs=pl.BlockSpec((bt, 128), lambda i: (i, 0)),
        compiler_params=pltpu.CompilerParams(
            dimension_semantics=("parallel",),
            vmem_limit_bytes=64 * 1024 * 1024),
    )(xc, e, wb1, b1t, wb2, b2t, g, fc1_b, fc2_w, fc2_b, fc3_w, fc3_b)
    return out[:b, :10]


_forward_jit = jax.jit(_forward)


def kernel(w1, b1, w2, b2, s2, fc1_w, fc1_b, fc2_w, fc2_b, fc3_w, fc3_b, x):
    return _forward_jit(w1, b1, w2, b2, s2, fc1_w, fc1_b, fc2_w, fc2_b,
                        fc3_w, fc3_b, x)


# speedup vs baseline: 23.1830x; 1.3033x over previous
import jax
import jax.numpy as jnp
from jax.experimental import pallas as pl
from jax.experimental.pallas import tpu as pltpu

_BT = 64


def _lenet_kernel(xr_ref, wb1_ref, b1t_ref, wb2_ref, b2t_ref, g_ref,
                  fc1b_ref, fc2w_ref, fc2b_ref, fc3w_ref, fc3b_ref,
                  out_ref):
    f32, bf16 = jnp.float32, jnp.bfloat16
    bt = xr_ref.shape[0]

    def shift_rows(x, k):
        return jnp.concatenate([x[:, k:, :], x[:, :k, :]], axis=1)

    def shift_lanes(x, k):
        return jnp.concatenate([x[:, :, k:], x[:, :, :k]], axis=2)

    x2 = jnp.pad(xr_ref[...], ((0, 0), (0, 0), (0, 32)))

    xbig = jnp.concatenate(
        [x2] + [shift_rows(x2, i) for i in range(1, 5)], axis=2)
    acc = jnp.dot(xbig.reshape(bt * 32, 640), wb1_ref[...],
                  preferred_element_type=f32)
    o1 = jnp.maximum(acc.reshape(bt, 32, 224) + b1t_ref[...], 0.0)

    u = jnp.maximum(o1, shift_rows(o1, 1))
    p1 = jnp.maximum(u, shift_lanes(u, 8)).astype(bf16)

    zpad = jnp.zeros((bt, 32, 32), bf16)
    slabs = []
    for i in range(5):
        slabs.append(shift_rows(p1, 2 * i) if i else p1)
        slabs.append(zpad)
    pbig = jnp.concatenate(slabs, axis=2)
    acc2 = jnp.dot(pbig.reshape(bt * 32, 1280), wb2_ref[...],
                   preferred_element_type=f32)
    o2 = jnp.maximum(acc2.reshape(bt, 32, 160) + b2t_ref[...], 0.0)

    u2 = jnp.maximum(o2, shift_rows(o2, 2))
    p2 = jnp.maximum(u2, shift_lanes(u2, 16)).astype(bf16)

    h = jnp.dot(p2[:, 0, :], g_ref[0], preferred_element_type=f32)
    for k in range(1, 5):
        h = h + jnp.dot(p2[:, 4 * k, :], g_ref[k], preferred_element_type=f32)
    h = jnp.maximum(h + fc1b_ref[...], 0.0)
    h = jnp.dot(h.astype(bf16), fc2w_ref[...], preferred_element_type=f32)
    h = jnp.maximum(h + fc2b_ref[...], 0.0)
    h = jnp.dot(h.astype(bf16), fc3w_ref[...], preferred_element_type=f32)
    out_ref[...] = h + fc3b_ref[...]


def _forward(w1, b1, w2, b2, s2, fc1_w, fc1_b, fc2_w, fc2_b, fc3_w, fc3_b, x):
    del s2
    f32, bf16 = jnp.float32, jnp.bfloat16
    b = x.shape[0]
    bt = _BT
    bp = ((b + bt - 1) // bt) * bt
    nb = bp // bt

    xin = jnp.transpose(x.astype(bf16), (0, 2, 1, 3)).reshape(b, 32, 96)
    if bp != b:
        xin = jnp.pad(xin, ((0, bp - b), (0, 0), (0, 0)))

    w1r = w1.astype(f32).reshape(5, 5, 3, 8)
    band1 = (jnp.arange(32)[None, :, None]
             == jnp.arange(28)[None, None, :]
             + jnp.arange(5)[:, None, None]).astype(f32)
    wb1 = jnp.einsum('ijco,jwp->icwpo', w1r, band1)
    wb1 = jnp.pad(wb1, ((0, 0), (0, 1), (0, 0), (0, 0), (0, 0)))
    wb1 = wb1.reshape(640, 224).astype(bf16)
    b1t = jnp.tile(b1, (1, 28))

    w2f = w2.astype(f32).reshape(5, 5, 8, 16)
    w2f = jnp.pad(w2f, ((0, 0), (0, 0), (0, 8), (0, 0)))
    band2 = (jnp.arange(14)[None, :, None]
             == jnp.arange(10)[None, None, :]
             + jnp.arange(5)[:, None, None]).astype(f32)
    wb2 = jnp.einsum('ijco,jwp->iwcpo', w2f, band2)
    wb2 = wb2.reshape(5, 224, 160)
    wb2 = jnp.pad(wb2, ((0, 0), (0, 32), (0, 0)))
    wb2 = wb2.reshape(1280, 160).astype(bf16)
    b2t = jnp.tile(b2, (1, 10))

    g = fc1_w[:400].reshape(5, 5, 16, 128)
    g = jnp.pad(g, ((0, 0), (0, 0), (0, 16), (0, 0))).reshape(5, 160, 128)

    c2 = lambda i: (0, 0)
    c3m = lambda i: (0, 0, 0)
    out = pl.pallas_call(
        _lenet_kernel,
        out_shape=jax.ShapeDtypeStruct((bp, 128), f32),
        grid=(nb,),
        in_specs=[
            pl.BlockSpec((bt, 32, 96), lambda i: (i, 0, 0)),
            pl.BlockSpec((640, 224), c2),
            pl.BlockSpec((1, 224), c2),
            pl.BlockSpec((1280, 160), c2),
            pl.BlockSpec((1, 160), c2),
            pl.BlockSpec((5, 160, 128), c3m),
            pl.BlockSpec((1, 128), c2),
            pl.BlockSpec((128, 128), c2),
            pl.BlockSpec((1, 128), c2),
            pl.BlockSpec((128, 128), c2),
            pl.BlockSpec((1, 128), c2),
        ],
        out_specs=pl.BlockSpec((bt, 128), lambda i: (i, 0)),
        compiler_params=pltpu.CompilerParams(
            dimension_semantics=("parallel",),
            vmem_limit_bytes=64 * 1024 * 1024),
    )(xin, wb1, b1t, wb2, b2t, g, fc1_b, fc2_w, fc2_b, fc3_w, fc3_b)
    return out[:b, :10]


_forward_jit = jax.jit(_forward)


def kernel(w1, b1, w2, b2, s2, fc1_w, fc1_b, fc2_w, fc2_b, fc3_w, fc3_b, x):
    return _forward_jit(w1, b1, w2, b2, s2, fc1_w, fc1_b, fc2_w, fc2_b,
                        fc3_w, fc3_b, x)


# speedup vs baseline: 23.3422x; 1.0069x over previous
import jax
import jax.numpy as jnp
from jax.experimental import pallas as pl
from jax.experimental.pallas import tpu as pltpu

_BT = 64


def _lenet_kernel(xr_ref, wb1_ref, wb2_ref, g_ref,
                  fc1b_ref, fc2w_ref, fc2b_ref, fc3w_ref, fc3b_ref,
                  out_ref):
    f32, bf16 = jnp.float32, jnp.bfloat16
    bt = xr_ref.shape[0]

    def shift_rows(x, k):
        return jnp.concatenate([x[:, k:, :], x[:, :k, :]], axis=1)

    def shift_lanes(x, k):
        return jnp.concatenate([x[:, :, k:], x[:, :, :k]], axis=2)

    xr = xr_ref[...]
    x2 = jnp.concatenate(
        [xr, jnp.ones((bt, 32, 1), bf16), jnp.zeros((bt, 32, 31), bf16)],
        axis=2)

    xbig = jnp.concatenate(
        [x2] + [shift_rows(x2, i) for i in range(1, 5)], axis=2)
    acc = jnp.dot(xbig.reshape(bt * 32, 640), wb1_ref[...],
                  preferred_element_type=f32)
    o1 = jnp.maximum(acc, 0.0).reshape(bt, 32, 256)

    u = jnp.maximum(o1, shift_rows(o1, 1))
    p1 = jnp.maximum(u, shift_lanes(u, 8)).astype(bf16)

    pbig = jnp.concatenate(
        [p1] + [shift_rows(p1, 2 * i) for i in range(1, 5)], axis=2)
    acc2 = jnp.dot(pbig.reshape(bt * 32, 1280), wb2_ref[...],
                   preferred_element_type=f32)
    o2 = jnp.maximum(acc2, 0.0).reshape(bt, 32, 160)

    u2 = jnp.maximum(o2, shift_rows(o2, 2))
    p2 = jnp.maximum(u2, shift_lanes(u2, 16)).astype(bf16)

    h = jnp.dot(p2[:, 0, :], g_ref[0], preferred_element_type=f32)
    for k in range(1, 5):
        h = h + jnp.dot(p2[:, 4 * k, :], g_ref[k], preferred_element_type=f32)
    h = jnp.maximum(h + fc1b_ref[...], 0.0)
    h = jnp.dot(h.astype(bf16), fc2w_ref[...], preferred_element_type=f32)
    h = jnp.maximum(h + fc2b_ref[...], 0.0)
    h = jnp.dot(h.astype(bf16), fc3w_ref[...], preferred_element_type=f32)
    out_ref[...] = h + fc3b_ref[...]


def _forward(w1, b1, w2, b2, s2, fc1_w, fc1_b, fc2_w, fc2_b, fc3_w, fc3_b, x):
    del s2
    f32, bf16 = jnp.float32, jnp.bfloat16
    b = x.shape[0]
    bt = _BT
    bp = ((b + bt - 1) // bt) * bt
    nb = bp // bt

    xin = jnp.transpose(x.astype(bf16), (0, 2, 1, 3)).reshape(b, 32, 96)
    if bp != b:
        xin = jnp.pad(xin, ((0, bp - b), (0, 0), (0, 0)))

    w1r = w1.astype(f32).reshape(5, 5, 3, 8)
    band1 = (jnp.arange(32)[None, :, None]
             == jnp.arange(28)[None, None, :]
             + jnp.arange(5)[:, None, None]).astype(f32)
    wb1 = jnp.einsum('ijco,jwp->icwpo', w1r, band1)
    wb1 = wb1.reshape(5, 3, 32, 224)
    wb1 = jnp.pad(wb1, ((0, 0), (0, 1), (0, 0), (0, 32)))
    brow1 = jnp.concatenate(
        [jnp.tile(b1[0], 28), jnp.ones((1,), f32), jnp.zeros((31,), f32)])
    wb1 = wb1.at[0, 3, 0, :].set(brow1)
    wb1 = wb1.reshape(640, 256).astype(bf16)

    w2f = w2.astype(f32).reshape(5, 5, 8, 16)
    w2f = jnp.pad(w2f, ((0, 0), (0, 0), (0, 8), (0, 0)))
    band2 = (jnp.arange(14)[None, :, None]
             == jnp.arange(10)[None, None, :]
             + jnp.arange(5)[:, None, None]).astype(f32)
    wb2 = jnp.einsum('ijco,jwp->iwcpo', w2f, band2)
    wb2 = wb2.reshape(5, 224, 160)
    wb2 = jnp.pad(wb2, ((0, 0), (0, 32), (0, 0)))
    wb2 = wb2.at[0, 224, :].set(jnp.tile(b2[0], 10))
    wb2 = wb2.reshape(1280, 160).astype(bf16)

    g = fc1_w[:400].reshape(5, 5, 16, 128)
    g = jnp.pad(g, ((0, 0), (0, 0), (0, 16), (0, 0))).reshape(5, 160, 128)

    c2 = lambda i: (0, 0)
    c3m = lambda i: (0, 0, 0)
    out = pl.pallas_call(
        _lenet_kernel,
        out_shape=jax.ShapeDtypeStruct((bp, 128), f32),
        grid=(nb,),
        in_specs=[
            pl.BlockSpec((bt, 32, 96), lambda i: (i, 0, 0)),
            pl.BlockSpec((640, 256), c2),
            pl.BlockSpec((1280, 160), c2),
            pl.BlockSpec((5, 160, 128), c3m),
            pl.BlockSpec((1, 128), c2),
            pl.BlockSpec((128, 128), c2),
            pl.BlockSpec((1, 128), c2),
            pl.BlockSpec((128, 128), c2),
            pl.BlockSpec((1, 128), c2),
        ],
        out_specs=pl.BlockSpec((bt, 128), lambda i: (i, 0)),
        compiler_params=pltpu.CompilerParams(
            dimension_semantics=("parallel",),
            vmem_limit_bytes=64 * 1024 * 1024),
    )(xin, wb1, wb2, g, fc1_b, fc2_w, fc2_b, fc3_w, fc3_b)
    return out[:b, :10]


_forward_jit = jax.jit(_forward)


def kernel(w1, b1, w2, b2, s2, fc1_w, fc1_b, fc2_w, fc2_b, fc3_w, fc3_b, x):
    return _forward_jit(w1, b1, w2, b2, s2, fc1_w, fc1_b, fc2_w, fc2_b,
                        fc3_w, fc3_b, x)
